# baseline (speedup 1.0000x reference)
import functools

import jax
import jax.numpy as jnp
from jax import lax
from jax.experimental import pallas as pl
from jax.experimental.pallas import tpu as pltpu
from jax.experimental.pallas import tpu_sc as plsc

FS = 0.09822694788464063
SCALE = 0.25 * FS

_NC = 2
_NS = 16
_NW = _NC * _NS

_C = 4000
_UNROLL = 25

_BL = 128000


def _sc_gather_recip(types, masses, off, h):
    n_chunks = h // _C
    assert h % _C == 0 and off % 8 == 0
    n_per = -(-n_chunks // _NW)

    mesh = plsc.VectorSubcoreMesh(core_axis_name="c", subcore_axis_name="s")

    @functools.partial(
        pl.kernel,
        out_type=jax.ShapeDtypeStruct((h,), jnp.float32),
        mesh=mesh,
        compiler_params=pltpu.CompilerParams(needs_layout_passes=False),
        scratch_types=[
            pltpu.VMEM((112,), jnp.float32),
            pltpu.VMEM((112,), jnp.float32),
            pltpu.VMEM((n_per * _C,), jnp.int32),
            pltpu.VMEM((n_per * _C,), jnp.float32),
            pltpu.SemaphoreType.DMA((n_per,)),
            pltpu.SemaphoreType.DMA((n_per,)),
        ],
    )
    def sc_kernel(types_hbm, masses_hbm, out_hbm, tab_v, recip_v, idx_v, res_v,
                  in_sems, out_sems):
        wid = lax.axis_index("s") * _NC + lax.axis_index("c")
        tab_v[pl.ds(96, 16)] = jnp.ones((16,), jnp.float32)
        pltpu.sync_copy(masses_hbm, tab_v.at[pl.ds(0, 100)])
        for j in range(112 // 16):
            recip_v[pl.ds(j * 16, 16)] = SCALE / tab_v[pl.ds(j * 16, 16)]

        cis = [jnp.minimum(wid + k * _NW, n_chunks - 1) for k in range(n_per)]

        for k in range(n_per):
            pltpu.async_copy(
                types_hbm.at[pl.ds(off + cis[k] * _C, _C)],
                idx_v.at[pl.ds(k * _C, _C)], in_sems.at[k])
        for k in range(n_per):
            pltpu.make_async_copy(
                types_hbm.at[pl.ds(off + cis[k] * _C, _C)],
                idx_v.at[pl.ds(k * _C, _C)], in_sems.at[k]).wait()

            def vec_body(i, _, k=k):
                o0 = i * (16 * _UNROLL)
                for j in range(_UNROLL):
                    o = o0 + j * 16
                    idx = idx_v[pl.ds(k * _C + o, 16)]
                    res_v[pl.ds(k * _C + o, 16)] = plsc.load_gather(recip_v, [idx])
                return 0

            lax.fori_loop(0, _C // (16 * _UNROLL), vec_body, 0)
            pltpu.async_copy(
                res_v.at[pl.ds(k * _C, _C)], out_hbm.at[pl.ds(cis[k] * _C, _C)],
                out_sems.at[k])
        for k in range(n_per):
            pltpu.make_async_copy(
                res_v.at[pl.ds(k * _C, _C)], out_hbm.at[pl.ds(cis[k] * _C, _C)],
                out_sems.at[k]).wait()

    return sc_kernel(types, masses)


def _tc_zero_fill(n, nblocks):
    def body(out_ref):
        out_ref[...] = jnp.zeros(out_ref.shape, jnp.float32)

    return pl.pallas_call(
        body,
        grid=(nblocks,),
        out_specs=pl.BlockSpec((4, 3, 1, _BL), lambda i: (0, 0, 0, i)),
        out_shape=jax.ShapeDtypeStruct((4, 3, 1, n), jnp.float32),
    )()


def _tc_scale(mom_t, rm, block_off, nblocks, order_dep=None, alias_out=None):
    n = mom_t.shape[-1]

    def body(mom_ref, rm_ref, *rest):
        out_ref = rest[-1]
        m = mom_ref[:, 0, :]
        r = rm_ref[...].reshape(1, -1)
        base = m * r
        base2 = base + base
        base4 = base2 + base2
        out_ref[0, :, 0, :] = base
        out_ref[1, :, 0, :] = base2
        out_ref[2, :, 0, :] = base4
        out_ref[3, :, 0, :] = base4 + base4

    in_specs = [
        pl.BlockSpec((3, 1, _BL), lambda i: (0, 0, i + block_off)),
        pl.BlockSpec((_BL,), lambda i: (i,)),
    ]
    args = [mom_t, rm]
    aliases = {}
    if order_dep is not None:
        in_specs.append(pl.BlockSpec(memory_space=pl.ANY))
        args.append(order_dep)
    if alias_out is not None:
        aliases = {len(args): 0}
        in_specs.append(pl.BlockSpec(memory_space=pl.ANY))
        args.append(alias_out)

    return pl.pallas_call(
        body,
        grid=(nblocks,),
        in_specs=in_specs,
        out_specs=pl.BlockSpec((4, 3, 1, _BL), lambda i: (0, 0, 0, i + block_off)),
        out_shape=jax.ShapeDtypeStruct((4, 3, 1, n), jnp.float32),
        input_output_aliases=aliases,
    )(*args)


def kernel(momenta, types, atomic_masses):
    n = types.shape[0]
    assert momenta.shape == (n, 3, 1)

    nblocks = -(-n // _BL)
    nb1 = nblocks // 2
    h1 = nb1 * _BL
    h2 = n - h1

    rm1 = _sc_gather_recip(types, atomic_masses, 0, h1)
    rm2 = _sc_gather_recip(types, atomic_masses, h1, h2)

    out_p = _tc_zero_fill(n, nblocks)
    mom_t = jnp.transpose(momenta, (1, 2, 0))
    out1 = _tc_scale(mom_t, rm1, 0, nb1, order_dep=out_p)
    out_q = _tc_scale(mom_t, rm2, nb1, nblocks - nb1, alias_out=out1)

    delta_q = jnp.transpose(out_q, (0, 3, 1, 2))
    delta_p = jnp.transpose(out_p, (0, 3, 1, 2))
    return (delta_p, delta_q)


# speedup vs baseline: 1.0506x; 1.0506x over previous
import functools

import jax
import jax.numpy as jnp
from jax import lax
from jax.experimental import pallas as pl
from jax.experimental.pallas import tpu as pltpu
from jax.experimental.pallas import tpu_sc as plsc

FS = 0.09822694788464063
SCALE = 0.25 * FS

_NC = 2
_NS = 16
_NW = _NC * _NS

_C = 2000
_UNROLL = 5

_BL = 128000


def _sc_gather_recip(types, masses, off, h):
    n_chunks = h // _C
    assert h % _C == 0 and off % 8 == 0
    n_per = -(-n_chunks // _NW)

    mesh = plsc.VectorSubcoreMesh(core_axis_name="c", subcore_axis_name="s")

    @functools.partial(
        pl.kernel,
        out_type=jax.ShapeDtypeStruct((h,), jnp.float32),
        mesh=mesh,
        compiler_params=pltpu.CompilerParams(needs_layout_passes=False),
        scratch_types=[
            pltpu.VMEM((112,), jnp.float32),
            pltpu.VMEM((112,), jnp.float32),
            pltpu.VMEM((n_per * _C,), jnp.int32),
            pltpu.VMEM((n_per * _C,), jnp.float32),
            pltpu.SemaphoreType.DMA((n_per,)),
            pltpu.SemaphoreType.DMA((n_per,)),
        ],
    )
    def sc_kernel(types_hbm, masses_hbm, out_hbm, tab_v, recip_v, idx_v, res_v,
                  in_sems, out_sems):
        wid = lax.axis_index("s") * _NC + lax.axis_index("c")
        tab_v[pl.ds(96, 16)] = jnp.ones((16,), jnp.float32)
        pltpu.sync_copy(masses_hbm, tab_v.at[pl.ds(0, 100)])
        for j in range(112 // 16):
            recip_v[pl.ds(j * 16, 16)] = SCALE / tab_v[pl.ds(j * 16, 16)]

        cis = [jnp.minimum(wid + k * _NW, n_chunks - 1) for k in range(n_per)]

        for k in range(n_per):
            pltpu.async_copy(
                types_hbm.at[pl.ds(off + cis[k] * _C, _C)],
                idx_v.at[pl.ds(k * _C, _C)], in_sems.at[k])
        for k in range(n_per):
            pltpu.make_async_copy(
                types_hbm.at[pl.ds(off + cis[k] * _C, _C)],
                idx_v.at[pl.ds(k * _C, _C)], in_sems.at[k]).wait()

            def vec_body(i, _, k=k):
                o0 = i * (16 * _UNROLL)
                for j in range(_UNROLL):
                    o = o0 + j * 16
                    idx = idx_v[pl.ds(k * _C + o, 16)]
                    res_v[pl.ds(k * _C + o, 16)] = plsc.load_gather(recip_v, [idx])
                return 0

            lax.fori_loop(0, _C // (16 * _UNROLL), vec_body, 0)
            pltpu.async_copy(
                res_v.at[pl.ds(k * _C, _C)], out_hbm.at[pl.ds(cis[k] * _C, _C)],
                out_sems.at[k])
        for k in range(n_per):
            pltpu.make_async_copy(
                res_v.at[pl.ds(k * _C, _C)], out_hbm.at[pl.ds(cis[k] * _C, _C)],
                out_sems.at[k]).wait()

    return sc_kernel(types, masses)


def _tc_zero_fill(n, nblocks):
    def body(out_ref):
        @pl.when(pl.program_id(0) < 3)
        def _():
            out_ref[...] = jnp.zeros(out_ref.shape, jnp.float32)

    return pl.pallas_call(
        body,
        grid=(nblocks,),
        out_specs=pl.BlockSpec((4, 3, 1, _BL), lambda i: (0, 0, 0, i)),
        out_shape=jax.ShapeDtypeStruct((4, 3, 1, n), jnp.float32),
    )()


def _tc_scale(mom_t, rm, block_off, nblocks, order_dep=None, alias_out=None):
    n = mom_t.shape[-1]

    def body(mom_ref, rm_ref, *rest):
        out_ref = rest[-1]
        m = mom_ref[:, 0, :]
        r = rm_ref[...].reshape(1, -1)
        base = m * r
        base2 = base + base
        base4 = base2 + base2
        out_ref[0, :, 0, :] = base
        out_ref[1, :, 0, :] = base2
        out_ref[2, :, 0, :] = base4
        out_ref[3, :, 0, :] = base4 + base4

    in_specs = [
        pl.BlockSpec((3, 1, _BL), lambda i: (0, 0, i + block_off)),
        pl.BlockSpec((_BL,), lambda i: (i,)),
    ]
    args = [mom_t, rm]
    aliases = {}
    if order_dep is not None:
        in_specs.append(pl.BlockSpec(memory_space=pl.ANY))
        args.append(order_dep)
    if alias_out is not None:
        aliases = {len(args): 0}
        in_specs.append(pl.BlockSpec(memory_space=pl.ANY))
        args.append(alias_out)

    return pl.pallas_call(
        body,
        grid=(nblocks,),
        in_specs=in_specs,
        out_specs=pl.BlockSpec((4, 3, 1, _BL), lambda i: (0, 0, 0, i + block_off)),
        out_shape=jax.ShapeDtypeStruct((4, 3, 1, n), jnp.float32),
        input_output_aliases=aliases,
    )(*args)


def kernel(momenta, types, atomic_masses):
    n = types.shape[0]
    assert momenta.shape == (n, 3, 1)

    nblocks = -(-n // _BL)
    nb1 = nblocks // 2
    h1 = nb1 * _BL
    h2 = n - h1

    rm1 = _sc_gather_recip(types, atomic_masses, 0, h1)
    rm2 = _sc_gather_recip(types, atomic_masses, h1, h2)

    out_p = _tc_zero_fill(n, nblocks)
    mom_t = jnp.transpose(momenta, (1, 2, 0))
    out1 = _tc_scale(mom_t, rm1, 0, nb1, order_dep=out_p)
    out_q = _tc_scale(mom_t, rm2, nb1, nblocks - nb1, alias_out=out1)

    delta_q = jnp.transpose(out_q, (0, 3, 1, 2))
    delta_p = jnp.transpose(out_p, (0, 3, 1, 2))
    return (delta_p, delta_q)


# speedup vs baseline: 1.0929x; 1.0403x over previous
import functools

import jax
import jax.numpy as jnp
from jax import lax
from jax.experimental import pallas as pl
from jax.experimental.pallas import tpu as pltpu
from jax.experimental.pallas import tpu_sc as plsc

FS = 0.09822694788464063
SCALE = 0.25 * FS

_NC = 2
_NS = 16
_NW = _NC * _NS

_C = 2000
_UNROLL = 5

_BL = 128000


def _sc_gather_recip(types, masses, off, h):
    n_chunks = h // _C
    assert h % _C == 0 and off % 8 == 0
    n_per = -(-n_chunks // _NW)

    mesh = plsc.VectorSubcoreMesh(core_axis_name="c", subcore_axis_name="s")

    @functools.partial(
        pl.kernel,
        out_type=jax.ShapeDtypeStruct((h,), jnp.float32),
        mesh=mesh,
        compiler_params=pltpu.CompilerParams(needs_layout_passes=False),
        scratch_types=[
            pltpu.VMEM((112,), jnp.float32),
            pltpu.VMEM((112,), jnp.float32),
            pltpu.VMEM((n_per * _C,), jnp.int32),
            pltpu.VMEM((n_per * _C,), jnp.float32),
            pltpu.SemaphoreType.DMA((n_per,)),
            pltpu.SemaphoreType.DMA((n_per,)),
        ],
    )
    def sc_kernel(types_hbm, masses_hbm, out_hbm, tab_v, recip_v, idx_v, res_v,
                  in_sems, out_sems):
        wid = lax.axis_index("s") * _NC + lax.axis_index("c")
        tab_v[pl.ds(96, 16)] = jnp.ones((16,), jnp.float32)
        pltpu.sync_copy(masses_hbm, tab_v.at[pl.ds(0, 100)])
        for j in range(112 // 16):
            recip_v[pl.ds(j * 16, 16)] = SCALE / tab_v[pl.ds(j * 16, 16)]

        cis = [jnp.minimum(wid + k * _NW, n_chunks - 1) for k in range(n_per)]

        for k in range(n_per):
            pltpu.async_copy(
                types_hbm.at[pl.ds(off + cis[k] * _C, _C)],
                idx_v.at[pl.ds(k * _C, _C)], in_sems.at[k])
        for k in range(n_per):
            pltpu.make_async_copy(
                types_hbm.at[pl.ds(off + cis[k] * _C, _C)],
                idx_v.at[pl.ds(k * _C, _C)], in_sems.at[k]).wait()

            def vec_body(i, _, k=k):
                o0 = i * (16 * _UNROLL)
                for j in range(_UNROLL):
                    o = o0 + j * 16
                    idx = idx_v[pl.ds(k * _C + o, 16)]
                    res_v[pl.ds(k * _C + o, 16)] = plsc.load_gather(recip_v, [idx])
                return 0

            lax.fori_loop(0, _C // (16 * _UNROLL), vec_body, 0)
            pltpu.async_copy(
                res_v.at[pl.ds(k * _C, _C)], out_hbm.at[pl.ds(cis[k] * _C, _C)],
                out_sems.at[k])
        for k in range(n_per):
            pltpu.make_async_copy(
                res_v.at[pl.ds(k * _C, _C)], out_hbm.at[pl.ds(cis[k] * _C, _C)],
                out_sems.at[k]).wait()

    return sc_kernel(types, masses)


def _tc_zero_fill(n, nblocks, order_dep):
    def body(dep_ref, out_ref):
        @pl.when(pl.program_id(0) < 3)
        def _():
            out_ref[...] = jnp.zeros(out_ref.shape, jnp.float32)

    return pl.pallas_call(
        body,
        grid=(nblocks,),
        in_specs=[pl.BlockSpec(memory_space=pl.ANY)],
        out_specs=pl.BlockSpec((4, 3, 1, _BL), lambda i: (0, 0, 0, i)),
        out_shape=jax.ShapeDtypeStruct((4, 3, 1, n), jnp.float32),
    )(order_dep)


def _tc_scale(mom_t, rm, block_off, nblocks, order_dep=None, alias_out=None):
    n = mom_t.shape[-1]

    def body(mom_ref, rm_ref, *rest):
        out_ref = rest[-1]
        m = mom_ref[:, 0, :]
        r = rm_ref[...].reshape(1, -1)
        base = m * r
        base2 = base + base
        base4 = base2 + base2
        out_ref[0, :, 0, :] = base
        out_ref[1, :, 0, :] = base2
        out_ref[2, :, 0, :] = base4
        out_ref[3, :, 0, :] = base4 + base4

    in_specs = [
        pl.BlockSpec((3, 1, _BL), lambda i: (0, 0, i + block_off)),
        pl.BlockSpec((_BL,), lambda i: (i,)),
    ]
    args = [mom_t, rm]
    aliases = {}
    if order_dep is not None:
        in_specs.append(pl.BlockSpec(memory_space=pl.ANY))
        args.append(order_dep)
    if alias_out is not None:
        aliases = {len(args): 0}
        in_specs.append(pl.BlockSpec(memory_space=pl.ANY))
        args.append(alias_out)

    return pl.pallas_call(
        body,
        grid=(nblocks,),
        in_specs=in_specs,
        out_specs=pl.BlockSpec((4, 3, 1, _BL), lambda i: (0, 0, 0, i + block_off)),
        out_shape=jax.ShapeDtypeStruct((4, 3, 1, n), jnp.float32),
        input_output_aliases=aliases,
    )(*args)


def kernel(momenta, types, atomic_masses):
    n = types.shape[0]
    assert momenta.shape == (n, 3, 1)

    nblocks = -(-n // _BL)
    nb1 = nblocks // 2
    h1 = nb1 * _BL
    h2 = n - h1

    rm1 = _sc_gather_recip(types, atomic_masses, 0, h1)
    rm2 = _sc_gather_recip(types, atomic_masses, h1, h2)

    out_p = _tc_zero_fill(n, nblocks, rm1)
    mom_t = jnp.transpose(momenta, (1, 2, 0))
    out1 = _tc_scale(mom_t, rm1, 0, nb1, order_dep=out_p)
    out_q = _tc_scale(mom_t, rm2, nb1, nblocks - nb1, alias_out=out1)

    delta_q = jnp.transpose(out_q, (0, 3, 1, 2))
    delta_p = jnp.transpose(out_p, (0, 3, 1, 2))
    return (delta_p, delta_q)


# speedup vs baseline: 1.1195x; 1.0244x over previous
import functools

import jax
import jax.numpy as jnp
from jax import lax
from jax.experimental import pallas as pl
from jax.experimental.pallas import tpu as pltpu
from jax.experimental.pallas import tpu_sc as plsc

FS = 0.09822694788464063
SCALE = 0.25 * FS

_NC = 2
_NS = 16
_NW = _NC * _NS

_C = 2000
_UNROLL = 5

_BL = 128000


def _sc_gather_recip(types, masses, off, h):
    n_chunks = h // _C
    assert h % _C == 0 and off % 8 == 0
    n_per = -(-n_chunks // _NW)

    mesh = plsc.VectorSubcoreMesh(core_axis_name="c", subcore_axis_name="s")

    @functools.partial(
        pl.kernel,
        out_type=jax.ShapeDtypeStruct((h,), jnp.float32),
        mesh=mesh,
        compiler_params=pltpu.CompilerParams(needs_layout_passes=False),
        scratch_types=[
            pltpu.VMEM((112,), jnp.float32),
            pltpu.VMEM((112,), jnp.float32),
            pltpu.VMEM((n_per * _C,), jnp.int32),
            pltpu.VMEM((n_per * _C,), jnp.float32),
            pltpu.SemaphoreType.DMA((n_per,)),
            pltpu.SemaphoreType.DMA((n_per,)),
        ],
    )
    def sc_kernel(types_hbm, masses_hbm, out_hbm, tab_v, recip_v, idx_v, res_v,
                  in_sems, out_sems):
        wid = lax.axis_index("s") * _NC + lax.axis_index("c")
        tab_v[pl.ds(96, 16)] = jnp.ones((16,), jnp.float32)
        pltpu.sync_copy(masses_hbm, tab_v.at[pl.ds(0, 100)])
        for j in range(112 // 16):
            recip_v[pl.ds(j * 16, 16)] = SCALE / tab_v[pl.ds(j * 16, 16)]

        cis = [jnp.minimum(wid + k * _NW, n_chunks - 1) for k in range(n_per)]

        for k in range(n_per):
            pltpu.async_copy(
                types_hbm.at[pl.ds(off + cis[k] * _C, _C)],
                idx_v.at[pl.ds(k * _C, _C)], in_sems.at[k])
        for k in range(n_per):
            pltpu.make_async_copy(
                types_hbm.at[pl.ds(off + cis[k] * _C, _C)],
                idx_v.at[pl.ds(k * _C, _C)], in_sems.at[k]).wait()

            def vec_body(i, _, k=k):
                o0 = i * (16 * _UNROLL)
                for j in range(_UNROLL):
                    o = o0 + j * 16
                    idx = idx_v[pl.ds(k * _C + o, 16)]
                    res_v[pl.ds(k * _C + o, 16)] = plsc.load_gather(recip_v, [idx])
                return 0

            lax.fori_loop(0, _C // (16 * _UNROLL), vec_body, 0)
            pltpu.async_copy(
                res_v.at[pl.ds(k * _C, _C)], out_hbm.at[pl.ds(cis[k] * _C, _C)],
                out_sems.at[k])
        for k in range(n_per):
            pltpu.make_async_copy(
                res_v.at[pl.ds(k * _C, _C)], out_hbm.at[pl.ds(cis[k] * _C, _C)],
                out_sems.at[k]).wait()

    return sc_kernel(types, masses)


def _tc_zero_fill(n, nblocks, order_dep):
    def body(dep_ref, out_ref):
        @pl.when(pl.program_id(0) < 3)
        def _():
            out_ref[...] = jnp.zeros(out_ref.shape, jnp.float32)

    return pl.pallas_call(
        body,
        grid=(nblocks,),
        in_specs=[pl.BlockSpec(memory_space=pl.ANY)],
        out_specs=pl.BlockSpec((4, 3, 1, _BL), lambda i: (0, 0, 0, i)),
        out_shape=jax.ShapeDtypeStruct((4, 3, 1, n), jnp.float32),
    )(order_dep)


def _tc_scale(mom_t, rm, block_off, nblocks, order_dep=None, alias_out=None):
    n = mom_t.shape[-1]

    def body(mom_ref, rm_ref, *rest):
        out_ref = rest[-1]
        m = mom_ref[:, 0, :]
        r = rm_ref[...].reshape(1, -1)
        base = m * r
        base2 = base + base
        base4 = base2 + base2
        out_ref[0, :, 0, :] = base
        out_ref[1, :, 0, :] = base2
        out_ref[2, :, 0, :] = base4
        out_ref[3, :, 0, :] = base4 + base4

    in_specs = [
        pl.BlockSpec((3, 1, _BL), lambda i: (0, 0, i + block_off)),
        pl.BlockSpec((_BL,), lambda i: (i,)),
    ]
    args = [mom_t, rm]
    aliases = {}
    if order_dep is not None:
        in_specs.append(pl.BlockSpec(memory_space=pl.ANY))
        args.append(order_dep)
    if alias_out is not None:
        aliases = {len(args): 0}
        in_specs.append(pl.BlockSpec(memory_space=pl.ANY))
        args.append(alias_out)

    return pl.pallas_call(
        body,
        grid=(nblocks,),
        in_specs=in_specs,
        out_specs=pl.BlockSpec((4, 3, 1, _BL), lambda i: (0, 0, 0, i + block_off)),
        out_shape=jax.ShapeDtypeStruct((4, 3, 1, n), jnp.float32),
        input_output_aliases=aliases,
    )(*args)


def kernel(momenta, types, atomic_masses):
    n = types.shape[0]
    assert momenta.shape == (n, 3, 1)

    nblocks = -(-n // _BL)
    nb1 = 3 if nblocks == 8 else nblocks // 2
    h1 = nb1 * _BL
    h2 = n - h1

    rm1 = _sc_gather_recip(types, atomic_masses, 0, h1)
    rm2 = _sc_gather_recip(types, atomic_masses, h1, h2)

    out_p = _tc_zero_fill(n, nblocks, rm1)
    mom_t = jnp.transpose(momenta, (1, 2, 0))
    out1 = _tc_scale(mom_t, rm1, 0, nb1, order_dep=out_p)
    out_q = _tc_scale(mom_t, rm2, nb1, nblocks - nb1, alias_out=out1)

    delta_q = jnp.transpose(out_q, (0, 3, 1, 2))
    delta_p = jnp.transpose(out_p, (0, 3, 1, 2))
    return (delta_p, delta_q)
